# baseline (speedup 1.0000x reference)
import jax
import jax.numpy as jnp
from jax import lax
from jax.experimental import pallas as pl
from jax.experimental.pallas import tpu as pltpu
from jax.experimental.pallas import tpu_sc as plsc

B = 1024
K = 16
D = 128
NW = 32
BPW = B // NW
SPW = BPW * K
HSPW = SPW // 2
L = 16


def _sc_body(u_h, v_h, adj_h, usr_h, ent_h,
             U_h, E0_h, E1_h, S2_h,
             vbuf, ubuf, vdiv8, vpad, adjv, e1idx, e1div8, e2big, e2idx,
             ent0, ent1, ent2, ent3, idx0, idx1, idx2, idx3,
             s2acc, zbuf, urows, e0rows,
             sem_u, sem_e0, sem_adj, sem_z, sem0, sem1, sem2, sem3,
             scsem0, scsem1, scsem2, scsem3):
    ents = (ent0, ent1, ent2, ent3)
    idxs = (idx0, idx1, idx2, idx3)
    sems = (sem0, sem1, sem2, sem3)
    scsems = (scsem0, scsem1, scsem2, scsem3)
    cid = lax.axis_index("c")
    sid = lax.axis_index("s")
    wid = sid * 2 + cid
    base = wid * BPW
    sbase = wid * SPW

    pltpu.sync_copy(v_h.at[pl.ds(base, BPW)], vbuf)
    pltpu.sync_copy(u_h.at[pl.ds(base, BPW)], ubuf)

    pltpu.async_copy(usr_h.at[ubuf], urows, sem_u)
    pltpu.async_copy(ent_h.at[vbuf], e0rows, sem_e0)

    zero = jnp.zeros((L,), jnp.float32)

    @pl.loop(0, 64)
    def _zero(r):
        for d in range(8):
            zbuf[r, pl.ds(d * L, L)] = zero

    for z in range(4):
        pltpu.async_copy(zbuf, s2acc.at[pl.ds(sid * HSPW + z * 64, 64)], sem_z)

    for g in range(BPW // L):
        vv = vbuf[pl.ds(g * L, L)]
        vdiv8[pl.ds(g * L, L)] = vv >> 3
        vpad[pl.ds(g * L, L)] = vv
    pltpu.async_copy(adj_h.at[vdiv8], adjv, sem_adj).wait()

    @pl.loop(0, BPW, unroll=16)
    def _extract1(r):
        off = (vpad[pl.ds(r, L)][0] & 7) * K
        e1idx[pl.ds(r * K, K)] = adjv[r, pl.ds(off, K)]

    for g in range(SPW // L):
        e1div8[pl.ds(g * L, L)] = e1idx[pl.ds(g * L, L)] >> 3

    pltpu.async_copy(adj_h.at[e1div8.at[pl.ds(0, 128)]], e2big, sem0)
    for c in range(4):
        pltpu.make_async_copy(adj_h.at[e1div8.at[pl.ds(c * 128, 128)]],
                              e2big, sem0).wait()

        @pl.loop(0, 128, unroll=32)
        def _extract2(r, c=c):
            p = c * 128 + r
            off = (e1idx[pl.ds(p, L)][0] & 7) * K
            e2idx[pl.ds(p * K, K)] = e2big[r, pl.ds(off, K)]

        if c < 3:
            pltpu.async_copy(adj_h.at[e1div8.at[pl.ds((c + 1) * 128, 128)]],
                             e2big, sem0)

    for c in range(4):
        pltpu.async_copy(ent_h.at[e1idx.at[pl.ds(c * 64, 64)]],
                         ents[c], sems[c])
    for c in range(8):
        j = c % 4
        pltpu.make_async_copy(ent_h.at[e1idx.at[pl.ds(0, 64)]],
                              ents[j], sems[j]).wait()
        pltpu.sync_copy(ents[j], E1_h.at[pl.ds(sbase + c * 64, 64)])
        if c < 4:
            pltpu.async_copy(ent_h.at[e1idx.at[pl.ds((c + 4) * 64, 64)]],
                             ents[j], sems[j])

    def _fire_gather(s, h):
        pltpu.async_copy(
            ent_h.at[e2idx.at[pl.ds((h * 64 + (s & 63)) * 64, 64)]],
            ents[s & 3], sems[s & 3])

    def _wait_gather(s):
        pltpu.make_async_copy(ent_h.at[e2idx.at[pl.ds(0, 64)]],
                              ents[s & 3], sems[s & 3]).wait()

    def _wait_gather_buf(b):
        pltpu.make_async_copy(ent_h.at[e2idx.at[pl.ds(0, 64)]],
                              ents[b], sems[b]).wait()

    for h in range(2):
        for z in range(4):
            pltpu.make_async_copy(
                zbuf, s2acc.at[pl.ds(sid * HSPW + z * 64, 64)], sem_z).wait()

        pltpu.async_copy(ent_h.at[e2idx.at[pl.ds(h * 64 * 64, 64)]],
                         ents[0], sems[0])
        pltpu.async_copy(ent_h.at[e2idx.at[pl.ds((h * 64 + 1) * 64, 64)]],
                         ents[1], sems[1])
        for s in range(4):
            sb2 = (s + 2) & 3
            for t in range(4):
                idxs[s][pl.ds(t * L, L)] = jnp.full(
                    (L,), sid * HSPW + s * 4 + t, jnp.int32)
            _wait_gather(s)
            pltpu.async_copy(ents[s], s2acc.at[idxs[s]], scsems[s], add=True)
            if s >= 2:
                pltpu.make_async_copy(ents[sb2], s2acc.at[idxs[sb2]],
                                      scsems[sb2]).wait()
            _fire_gather(s + 2, h)

        @pl.loop(0, 15)
        def _hop2(ii, h=h):
            for j in range(4):
                s = (ii + 1) * 4 + j
                sb2 = (j + 2) & 3
                for t in range(4):
                    idxs[j][pl.ds(t * L, L)] = jnp.full(
                        (L,), sid * HSPW + s * 4 + t, jnp.int32)
                _wait_gather_buf(j)
                pltpu.async_copy(ents[j], s2acc.at[idxs[j]], scsems[j],
                                 add=True)
                pltpu.make_async_copy(ents[sb2], s2acc.at[idxs[sb2]],
                                      scsems[sb2]).wait()
                pltpu.async_copy(
                    ent_h.at[e2idx.at[pl.ds(
                        (h * 64 + ((s + 2) & 63)) * 64, 64)]],
                    ents[sb2], sems[sb2])

        for s in (62, 63):
            pltpu.make_async_copy(ents[s & 3], s2acc.at[idxs[s & 3]],
                                  scsems[s & 3]).wait()
            _wait_gather(s + 2)

        pltpu.sync_copy(s2acc.at[pl.ds(sid * HSPW, HSPW)],
                        S2_h.at[pl.ds(sbase + h * HSPW, HSPW)])

        if h == 0:
            for z in range(4):
                pltpu.async_copy(
                    zbuf, s2acc.at[pl.ds(sid * HSPW + z * 64, 64)], sem_z)

    pltpu.make_async_copy(usr_h.at[ubuf], urows, sem_u).wait()
    pltpu.sync_copy(urows, U_h.at[pl.ds(base, BPW)])
    pltpu.make_async_copy(ent_h.at[vbuf], e0rows, sem_e0).wait()
    pltpu.sync_copy(e0rows, E0_h.at[pl.ds(base, BPW)])


def _sc_gather(u, v, adj128, usr_table, ent_table):
    mesh = plsc.VectorSubcoreMesh(core_axis_name="c", subcore_axis_name="s")
    f32 = jnp.float32
    kern = pl.kernel(
        _sc_body,
        out_type=(
            jax.ShapeDtypeStruct((B, D), f32),
            jax.ShapeDtypeStruct((B, D), f32),
            jax.ShapeDtypeStruct((B * K, D), f32),
            jax.ShapeDtypeStruct((B * K, D), f32),
        ),
        mesh=mesh,
        scratch_types=[
            pltpu.VMEM((BPW,), jnp.int32),
            pltpu.VMEM((BPW,), jnp.int32),
            pltpu.VMEM((BPW,), jnp.int32),
            pltpu.VMEM((BPW + L,), jnp.int32),
            pltpu.VMEM((BPW, 128), jnp.int32),
            pltpu.VMEM((SPW + L,), jnp.int32),
            pltpu.VMEM((SPW,), jnp.int32),
            pltpu.VMEM((128, 128), jnp.int32),
            pltpu.VMEM((SPW * K,), jnp.int32),
            pltpu.VMEM((64, D), f32),
            pltpu.VMEM((64, D), f32),
            pltpu.VMEM((64, D), f32),
            pltpu.VMEM((64, D), f32),
            pltpu.VMEM((64,), jnp.int32),
            pltpu.VMEM((64,), jnp.int32),
            pltpu.VMEM((64,), jnp.int32),
            pltpu.VMEM((64,), jnp.int32),
            pltpu.VMEM_SHARED((16 * HSPW, D), f32),
            pltpu.VMEM((64, D), f32),
            pltpu.VMEM((BPW, D), f32),
            pltpu.VMEM((BPW, D), f32),
            pltpu.SemaphoreType.DMA,
            pltpu.SemaphoreType.DMA,
            pltpu.SemaphoreType.DMA,
            pltpu.SemaphoreType.DMA,
            pltpu.SemaphoreType.DMA,
            pltpu.SemaphoreType.DMA,
            pltpu.SemaphoreType.DMA,
            pltpu.SemaphoreType.DMA,
            pltpu.SemaphoreType.DMA,
            pltpu.SemaphoreType.DMA,
            pltpu.SemaphoreType.DMA,
            pltpu.SemaphoreType.DMA,
        ],
    )
    return kern(u, v, adj128, usr_table, ent_table)


def _tc_body(u_ref, e0_ref, e1_ref, s2_ref, w0_ref, b0_ref, w1_ref, b1_ref,
             out_ref):
    f32 = jnp.float32
    bb = e0_ref.shape[0]
    w0 = w0_ref[...]
    b0 = b0_ref[...]
    comb1 = e1_ref[...] + s2_ref[...] * (1.0 / K)
    x1 = jnp.maximum(jnp.dot(comb1, w0, preferred_element_type=f32) + b0, 0.0)
    m0 = jnp.mean(e1_ref[...].reshape(bb, K, D), axis=1)
    x0 = jnp.maximum(
        jnp.dot(e0_ref[...] + m0, w0, preferred_element_type=f32) + b0, 0.0)
    m1 = jnp.mean(x1.reshape(bb, K, D), axis=1)
    item = jnp.tanh(
        jnp.dot(x0 + m1, w1_ref[...], preferred_element_type=f32) + b1_ref[...])
    out_ref[...] = jnp.sum(u_ref[...] * item, axis=1)


def _tc_dense(U, E0, E1, S2, W0, b0, W1, b1):
    BB = 128
    grid = B // BB
    return pl.pallas_call(
        _tc_body,
        grid=(grid,),
        in_specs=[
            pl.BlockSpec((BB, D), lambda i: (i, 0)),
            pl.BlockSpec((BB, D), lambda i: (i, 0)),
            pl.BlockSpec((BB * K, D), lambda i: (i, 0)),
            pl.BlockSpec((BB * K, D), lambda i: (i, 0)),
            pl.BlockSpec((D, D), lambda i: (0, 0)),
            pl.BlockSpec((1, D), lambda i: (0, 0)),
            pl.BlockSpec((D, D), lambda i: (0, 0)),
            pl.BlockSpec((1, D), lambda i: (0, 0)),
        ],
        out_specs=pl.BlockSpec((BB,), lambda i: (i,)),
        out_shape=jax.ShapeDtypeStruct((B,), jnp.float32),
    )(U, E0, E1, S2, W0, b0, W1, b1)


def kernel(u, v, adj, rel, usr_table, ent_table, rel_table, W0, b0, W1, b1):
    del rel, rel_table
    u = u.astype(jnp.int32)
    v = v.astype(jnp.int32)
    adj128 = adj.astype(jnp.int32).reshape(-1, 128)
    U, E0, E1, S2 = _sc_gather(u, v, adj128, usr_table, ent_table)
    return _tc_dense(U, E0, E1, S2, W0, b0.reshape(1, D), W1, b1.reshape(1, D))


# speedup vs baseline: 1.0652x; 1.0652x over previous
import jax
import jax.numpy as jnp
from jax import lax
from jax.experimental import pallas as pl
from jax.experimental.pallas import tpu as pltpu
from jax.experimental.pallas import tpu_sc as plsc

B = 1024
K = 16
D = 128
NW = 32
BPW = B // NW
SPW = BPW * K
HSPW = SPW // 2
L = 16


def _sc_body(u_h, v_h, adj_h, usr_h, ent_h,
             U_h, E0_h, E1_h, S2_h,
             vbuf, ubuf, vdiv8, vpad, adjv, e1idx, e1div8, e2big, e2idx,
             ent0, ent1, ent2, ent3, idx0, idx1, idx2, idx3,
             s2acc, zbuf, urows, e0rows,
             sem_u, sem_e0, sem_adj, sem_z, sem0, sem1, sem2, sem3):
    ents = (ent0, ent1, ent2, ent3)
    idxs = (idx0, idx1, idx2, idx3)
    sems = (sem0, sem1, sem2, sem3)
    cid = lax.axis_index("c")
    sid = lax.axis_index("s")
    wid = sid * 2 + cid
    base = wid * BPW
    sbase = wid * SPW

    pltpu.sync_copy(v_h.at[pl.ds(base, BPW)], vbuf)
    pltpu.sync_copy(u_h.at[pl.ds(base, BPW)], ubuf)

    pltpu.async_copy(usr_h.at[ubuf], urows, sem_u)
    pltpu.async_copy(ent_h.at[vbuf], e0rows, sem_e0)

    zero = jnp.zeros((L,), jnp.float32)

    @pl.loop(0, 64)
    def _zero(r):
        for d in range(8):
            zbuf[r, pl.ds(d * L, L)] = zero

    for z in range(4):
        pltpu.async_copy(zbuf, s2acc.at[pl.ds(sid * HSPW + z * 64, 64)], sem_z)

    for g in range(BPW // L):
        vv = vbuf[pl.ds(g * L, L)]
        vdiv8[pl.ds(g * L, L)] = vv >> 3
        vpad[pl.ds(g * L, L)] = vv
    pltpu.async_copy(adj_h.at[vdiv8], adjv, sem_adj).wait()

    @pl.loop(0, BPW, unroll=16)
    def _extract1(r):
        off = (vpad[pl.ds(r, L)][0] & 7) * K
        e1idx[pl.ds(r * K, K)] = adjv[r, pl.ds(off, K)]

    for g in range(SPW // L):
        e1div8[pl.ds(g * L, L)] = e1idx[pl.ds(g * L, L)] >> 3

    pltpu.async_copy(adj_h.at[e1div8.at[pl.ds(0, 128)]], e2big, sem0)
    for c in range(4):
        pltpu.make_async_copy(adj_h.at[e1div8.at[pl.ds(c * 128, 128)]],
                              e2big, sem0).wait()

        @pl.loop(0, 128, unroll=32)
        def _extract2(r, c=c):
            p = c * 128 + r
            off = (e1idx[pl.ds(p, L)][0] & 7) * K
            e2idx[pl.ds(p * K, K)] = e2big[r, pl.ds(off, K)]

        if c < 3:
            pltpu.async_copy(adj_h.at[e1div8.at[pl.ds((c + 1) * 128, 128)]],
                             e2big, sem0)

    for c in range(4):
        pltpu.async_copy(ent_h.at[e1idx.at[pl.ds(c * 64, 64)]],
                         ents[c], sems[c])
    for c in range(8):
        j = c % 4
        pltpu.make_async_copy(ent_h.at[e1idx.at[pl.ds(0, 64)]],
                              ents[j], sems[j]).wait()
        pltpu.sync_copy(ents[j], E1_h.at[pl.ds(sbase + c * 64, 64)])
        if c < 4:
            pltpu.async_copy(ent_h.at[e1idx.at[pl.ds((c + 4) * 64, 64)]],
                             ents[j], sems[j])

    for h in range(2):
        for z in range(4):
            pltpu.make_async_copy(
                zbuf, s2acc.at[pl.ds(sid * HSPW + z * 64, 64)], sem_z).wait()

        for j in range(4):
            pltpu.async_copy(
                ent_h.at[e2idx.at[pl.ds((h * 64 + j) * 64, 64)]],
                ents[j], sems[j])

        @pl.loop(0, 16)
        def _hop2(i, h=h):
            for j in range(4):
                tbase = sid * HSPW + i * 16 + j * 4
                for t in range(4):
                    idxs[j][pl.ds(t * L, L)] = jnp.full((L,), tbase + t,
                                                        jnp.int32)
                pltpu.make_async_copy(ent_h.at[e2idx.at[pl.ds(0, 64)]],
                                      ents[j], sems[j]).wait()
                pltpu.sync_copy(ents[j], s2acc.at[idxs[j]], add=True)
                pltpu.async_copy(
                    ent_h.at[
                        e2idx.at[pl.ds((h * 64 + ((4 * i + j + 4) & 63)) * 64,
                                       64)]],
                    ents[j], sems[j])

        for j in range(4):
            pltpu.make_async_copy(ent_h.at[e2idx.at[pl.ds(0, 64)]],
                                  ents[j], sems[j]).wait()

        pltpu.sync_copy(s2acc.at[pl.ds(sid * HSPW, HSPW)],
                        S2_h.at[pl.ds(sbase + h * HSPW, HSPW)])

        if h == 0:
            for z in range(4):
                pltpu.async_copy(
                    zbuf, s2acc.at[pl.ds(sid * HSPW + z * 64, 64)], sem_z)

    pltpu.make_async_copy(usr_h.at[ubuf], urows, sem_u).wait()
    pltpu.sync_copy(urows, U_h.at[pl.ds(base, BPW)])
    pltpu.make_async_copy(ent_h.at[vbuf], e0rows, sem_e0).wait()
    pltpu.sync_copy(e0rows, E0_h.at[pl.ds(base, BPW)])


def _sc_gather(u, v, adj128, usr_table, ent_table):
    mesh = plsc.VectorSubcoreMesh(core_axis_name="c", subcore_axis_name="s")
    f32 = jnp.float32
    kern = pl.kernel(
        _sc_body,
        out_type=(
            jax.ShapeDtypeStruct((B, D), f32),
            jax.ShapeDtypeStruct((B, D), f32),
            jax.ShapeDtypeStruct((B * K, D), f32),
            jax.ShapeDtypeStruct((B * K, D), f32),
        ),
        mesh=mesh,
        scratch_types=[
            pltpu.VMEM((BPW,), jnp.int32),
            pltpu.VMEM((BPW,), jnp.int32),
            pltpu.VMEM((BPW,), jnp.int32),
            pltpu.VMEM((BPW + L,), jnp.int32),
            pltpu.VMEM((BPW, 128), jnp.int32),
            pltpu.VMEM((SPW + L,), jnp.int32),
            pltpu.VMEM((SPW,), jnp.int32),
            pltpu.VMEM((128, 128), jnp.int32),
            pltpu.VMEM((SPW * K,), jnp.int32),
            pltpu.VMEM((64, D), f32),
            pltpu.VMEM((64, D), f32),
            pltpu.VMEM((64, D), f32),
            pltpu.VMEM((64, D), f32),
            pltpu.VMEM((64,), jnp.int32),
            pltpu.VMEM((64,), jnp.int32),
            pltpu.VMEM((64,), jnp.int32),
            pltpu.VMEM((64,), jnp.int32),
            pltpu.VMEM_SHARED((16 * HSPW, D), f32),
            pltpu.VMEM((64, D), f32),
            pltpu.VMEM((BPW, D), f32),
            pltpu.VMEM((BPW, D), f32),
            pltpu.SemaphoreType.DMA,
            pltpu.SemaphoreType.DMA,
            pltpu.SemaphoreType.DMA,
            pltpu.SemaphoreType.DMA,
            pltpu.SemaphoreType.DMA,
            pltpu.SemaphoreType.DMA,
            pltpu.SemaphoreType.DMA,
            pltpu.SemaphoreType.DMA,
        ],
    )
    return kern(u, v, adj128, usr_table, ent_table)


def _tc_body(u_ref, e0_ref, e1_ref, s2_ref, w0_ref, b0_ref, w1_ref, b1_ref,
             out_ref):
    f32 = jnp.float32
    bb = e0_ref.shape[0]
    w0 = w0_ref[...]
    b0 = b0_ref[...]
    comb1 = e1_ref[...] + s2_ref[...] * (1.0 / K)
    x1 = jnp.maximum(jnp.dot(comb1, w0, preferred_element_type=f32) + b0, 0.0)
    m0 = jnp.mean(e1_ref[...].reshape(bb, K, D), axis=1)
    x0 = jnp.maximum(
        jnp.dot(e0_ref[...] + m0, w0, preferred_element_type=f32) + b0, 0.0)
    m1 = jnp.mean(x1.reshape(bb, K, D), axis=1)
    item = jnp.tanh(
        jnp.dot(x0 + m1, w1_ref[...], preferred_element_type=f32) + b1_ref[...])
    out_ref[...] = jnp.sum(u_ref[...] * item, axis=1)


def _tc_dense(U, E0, E1, S2, W0, b0, W1, b1):
    BB = 128
    grid = B // BB
    return pl.pallas_call(
        _tc_body,
        grid=(grid,),
        in_specs=[
            pl.BlockSpec((BB, D), lambda i: (i, 0)),
            pl.BlockSpec((BB, D), lambda i: (i, 0)),
            pl.BlockSpec((BB * K, D), lambda i: (i, 0)),
            pl.BlockSpec((BB * K, D), lambda i: (i, 0)),
            pl.BlockSpec((D, D), lambda i: (0, 0)),
            pl.BlockSpec((1, D), lambda i: (0, 0)),
            pl.BlockSpec((D, D), lambda i: (0, 0)),
            pl.BlockSpec((1, D), lambda i: (0, 0)),
        ],
        out_specs=pl.BlockSpec((BB,), lambda i: (i,)),
        out_shape=jax.ShapeDtypeStruct((B,), jnp.float32),
    )(U, E0, E1, S2, W0, b0, W1, b1)


def kernel(u, v, adj, rel, usr_table, ent_table, rel_table, W0, b0, W1, b1):
    del rel, rel_table
    u = u.astype(jnp.int32)
    v = v.astype(jnp.int32)
    adj128 = adj.astype(jnp.int32).reshape(-1, 128)
    U, E0, E1, S2 = _sc_gather(u, v, adj128, usr_table, ent_table)
    return _tc_dense(U, E0, E1, S2, W0, b0.reshape(1, D), W1, b1.reshape(1, D))


# speedup vs baseline: 1.0711x; 1.0056x over previous
import jax
import jax.numpy as jnp
from jax import lax
from jax.experimental import pallas as pl
from jax.experimental.pallas import tpu as pltpu
from jax.experimental.pallas import tpu_sc as plsc

B = 1024
K = 16
D = 128
NW = 32
BPW = B // NW
SPW = BPW * K
HSPW = SPW // 2
L = 16


def _sc_body(u_h, v_h, adj_h, usr_h, ent_h,
             U_h, E0_h, E1_h, S2_h,
             vbuf, ubuf, vdiv8, vpad, adjv, e1idx, e1div8, e2big, e2idx,
             ent0, ent1, ent2, ent3, idx0, idx1, idx2, idx3,
             s2acc, zbuf, urows, e0rows,
             sem_u, sem_e0, sem_adj, sem_z, sem0, sem1, sem2, sem3):
    ents = (ent0, ent1, ent2, ent3)
    idxs = (idx0, idx1, idx2, idx3)
    sems = (sem0, sem1, sem2, sem3)
    cid = lax.axis_index("c")
    sid = lax.axis_index("s")
    wid = sid * 2 + cid
    base = wid * BPW
    sbase = wid * SPW

    pltpu.sync_copy(v_h.at[pl.ds(base, BPW)], vbuf)
    pltpu.sync_copy(u_h.at[pl.ds(base, BPW)], ubuf)

    pltpu.async_copy(usr_h.at[ubuf], urows, sem_u)
    pltpu.async_copy(ent_h.at[vbuf], e0rows, sem_e0)

    zero = jnp.zeros((L,), jnp.float32)

    @pl.loop(0, 64)
    def _zero(r):
        for d in range(8):
            zbuf[r, pl.ds(d * L, L)] = zero

    for z in range(4):
        pltpu.async_copy(zbuf, s2acc.at[pl.ds(sid * HSPW + z * 64, 64)], sem_z)

    for g in range(BPW // L):
        vv = vbuf[pl.ds(g * L, L)]
        vdiv8[pl.ds(g * L, L)] = vv >> 3
        vpad[pl.ds(g * L, L)] = vv
    pltpu.async_copy(adj_h.at[vdiv8], adjv, sem_adj).wait()

    @pl.loop(0, BPW, unroll=8)
    def _extract1(r):
        off = (vpad[pl.ds(r, L)][0] & 7) * K
        e1idx[pl.ds(r * K, K)] = adjv[r, pl.ds(off, K)]

    for g in range(SPW // L):
        e1div8[pl.ds(g * L, L)] = e1idx[pl.ds(g * L, L)] >> 3

    pltpu.async_copy(adj_h.at[e1div8.at[pl.ds(0, 128)]], e2big, sem0)
    for c in range(4):
        pltpu.make_async_copy(adj_h.at[e1div8.at[pl.ds(c * 128, 128)]],
                              e2big, sem0).wait()

        @pl.loop(0, 128, unroll=16)
        def _extract2(r, c=c):
            p = c * 128 + r
            off = (e1idx[pl.ds(p, L)][0] & 7) * K
            e2idx[pl.ds(p * K, K)] = e2big[r, pl.ds(off, K)]

        if c < 3:
            pltpu.async_copy(adj_h.at[e1div8.at[pl.ds((c + 1) * 128, 128)]],
                             e2big, sem0)

    for c in range(4):
        pltpu.async_copy(ent_h.at[e1idx.at[pl.ds(c * 64, 64)]],
                         ents[c], sems[c])
    for c in range(8):
        j = c % 4
        pltpu.make_async_copy(ent_h.at[e1idx.at[pl.ds(0, 64)]],
                              ents[j], sems[j]).wait()
        pltpu.sync_copy(ents[j], E1_h.at[pl.ds(sbase + c * 64, 64)])
        if c < 4:
            pltpu.async_copy(ent_h.at[e1idx.at[pl.ds((c + 4) * 64, 64)]],
                             ents[j], sems[j])

    for h in range(2):
        for z in range(4):
            pltpu.make_async_copy(
                zbuf, s2acc.at[pl.ds(sid * HSPW + z * 64, 64)], sem_z).wait()

        for j in range(4):
            pltpu.async_copy(
                ent_h.at[e2idx.at[pl.ds((h * 64 + j) * 64, 64)]],
                ents[j], sems[j])

        @pl.loop(0, 16)
        def _hop2(i, h=h):
            for j in range(4):
                tbase = sid * HSPW + i * 16 + j * 4
                for t in range(4):
                    idxs[j][pl.ds(t * L, L)] = jnp.full((L,), tbase + t,
                                                        jnp.int32)
                pltpu.make_async_copy(ent_h.at[e2idx.at[pl.ds(0, 64)]],
                                      ents[j], sems[j]).wait()
                pltpu.sync_copy(ents[j], s2acc.at[idxs[j]], add=True)
                pltpu.async_copy(
                    ent_h.at[
                        e2idx.at[pl.ds((h * 64 + ((4 * i + j + 4) & 63)) * 64,
                                       64)]],
                    ents[j], sems[j])

        for j in range(4):
            pltpu.make_async_copy(ent_h.at[e2idx.at[pl.ds(0, 64)]],
                                  ents[j], sems[j]).wait()

        pltpu.sync_copy(s2acc.at[pl.ds(sid * HSPW, HSPW)],
                        S2_h.at[pl.ds(sbase + h * HSPW, HSPW)])

        if h == 0:
            for z in range(4):
                pltpu.async_copy(
                    zbuf, s2acc.at[pl.ds(sid * HSPW + z * 64, 64)], sem_z)

    pltpu.make_async_copy(usr_h.at[ubuf], urows, sem_u).wait()
    pltpu.sync_copy(urows, U_h.at[pl.ds(base, BPW)])
    pltpu.make_async_copy(ent_h.at[vbuf], e0rows, sem_e0).wait()
    pltpu.sync_copy(e0rows, E0_h.at[pl.ds(base, BPW)])


def _sc_gather(u, v, adj128, usr_table, ent_table):
    mesh = plsc.VectorSubcoreMesh(core_axis_name="c", subcore_axis_name="s")
    f32 = jnp.float32
    kern = pl.kernel(
        _sc_body,
        out_type=(
            jax.ShapeDtypeStruct((B, D), f32),
            jax.ShapeDtypeStruct((B, D), f32),
            jax.ShapeDtypeStruct((B * K, D), f32),
            jax.ShapeDtypeStruct((B * K, D), f32),
        ),
        mesh=mesh,
        scratch_types=[
            pltpu.VMEM((BPW,), jnp.int32),
            pltpu.VMEM((BPW,), jnp.int32),
            pltpu.VMEM((BPW,), jnp.int32),
            pltpu.VMEM((BPW + L,), jnp.int32),
            pltpu.VMEM((BPW, 128), jnp.int32),
            pltpu.VMEM((SPW + L,), jnp.int32),
            pltpu.VMEM((SPW,), jnp.int32),
            pltpu.VMEM((128, 128), jnp.int32),
            pltpu.VMEM((SPW * K,), jnp.int32),
            pltpu.VMEM((64, D), f32),
            pltpu.VMEM((64, D), f32),
            pltpu.VMEM((64, D), f32),
            pltpu.VMEM((64, D), f32),
            pltpu.VMEM((64,), jnp.int32),
            pltpu.VMEM((64,), jnp.int32),
            pltpu.VMEM((64,), jnp.int32),
            pltpu.VMEM((64,), jnp.int32),
            pltpu.VMEM_SHARED((16 * HSPW, D), f32),
            pltpu.VMEM((64, D), f32),
            pltpu.VMEM((BPW, D), f32),
            pltpu.VMEM((BPW, D), f32),
            pltpu.SemaphoreType.DMA,
            pltpu.SemaphoreType.DMA,
            pltpu.SemaphoreType.DMA,
            pltpu.SemaphoreType.DMA,
            pltpu.SemaphoreType.DMA,
            pltpu.SemaphoreType.DMA,
            pltpu.SemaphoreType.DMA,
            pltpu.SemaphoreType.DMA,
        ],
    )
    return kern(u, v, adj128, usr_table, ent_table)


def _tc_body(u_ref, e0_ref, e1_ref, s2_ref, w0_ref, b0_ref, w1_ref, b1_ref,
             out_ref):
    f32 = jnp.float32
    bb = e0_ref.shape[0]
    w0 = w0_ref[...]
    b0 = b0_ref[...]
    comb1 = e1_ref[...] + s2_ref[...] * (1.0 / K)
    x1 = jnp.maximum(jnp.dot(comb1, w0, preferred_element_type=f32) + b0, 0.0)
    m0 = jnp.mean(e1_ref[...].reshape(bb, K, D), axis=1)
    x0 = jnp.maximum(
        jnp.dot(e0_ref[...] + m0, w0, preferred_element_type=f32) + b0, 0.0)
    m1 = jnp.mean(x1.reshape(bb, K, D), axis=1)
    item = jnp.tanh(
        jnp.dot(x0 + m1, w1_ref[...], preferred_element_type=f32) + b1_ref[...])
    out_ref[...] = jnp.sum(u_ref[...] * item, axis=1)


def _tc_dense(U, E0, E1, S2, W0, b0, W1, b1):
    BB = 128
    grid = B // BB
    return pl.pallas_call(
        _tc_body,
        grid=(grid,),
        in_specs=[
            pl.BlockSpec((BB, D), lambda i: (i, 0)),
            pl.BlockSpec((BB, D), lambda i: (i, 0)),
            pl.BlockSpec((BB * K, D), lambda i: (i, 0)),
            pl.BlockSpec((BB * K, D), lambda i: (i, 0)),
            pl.BlockSpec((D, D), lambda i: (0, 0)),
            pl.BlockSpec((1, D), lambda i: (0, 0)),
            pl.BlockSpec((D, D), lambda i: (0, 0)),
            pl.BlockSpec((1, D), lambda i: (0, 0)),
        ],
        out_specs=pl.BlockSpec((BB,), lambda i: (i,)),
        out_shape=jax.ShapeDtypeStruct((B,), jnp.float32),
    )(U, E0, E1, S2, W0, b0, W1, b1)


def kernel(u, v, adj, rel, usr_table, ent_table, rel_table, W0, b0, W1, b1):
    del rel, rel_table
    u = u.astype(jnp.int32)
    v = v.astype(jnp.int32)
    adj128 = adj.astype(jnp.int32).reshape(-1, 128)
    U, E0, E1, S2 = _sc_gather(u, v, adj128, usr_table, ent_table)
    return _tc_dense(U, E0, E1, S2, W0, b0.reshape(1, D), W1, b1.reshape(1, D))
